# Initial kernel scaffold; baseline (speedup 1.0000x reference)
import jax
import jax.numpy as jnp
from jax.experimental import pallas as pl


def kernel(query):
    raise NotImplementedError("write your pallas kernel here")


# speedup vs baseline: 174.8504x; 174.8504x over previous
import functools

import jax
import jax.numpy as jnp
import numpy as np
from jax import lax
from jax.experimental import pallas as pl
from jax.experimental.pallas import tpu as pltpu
from jax.experimental.pallas import tpu_sc as plsc

N = 16384
W = 2048
D = 128
J = W // D
NC, NS = 2, 16
NW = NC * NS
ROWS_PER_W = N // NW
CH = 32
NCH = ROWS_PER_W // CH


def _build_indices():
    rng = np.random.RandomState(0)
    perm_p = rng.permutation(N)
    x_p = int(rng.randint(W))
    perm_n = rng.permutation(N)
    x_n = int(rng.randint(W))
    x1p = int(np.clip(x_p - 300, 0, W))
    x2p = int(np.clip(x_p + 300, 0, W))
    x1n = int(np.clip(x_n - 500, 0, W))
    x2n = int(np.clip(x_n + 500, 0, W))
    assert (x1n, x2n, x1p, x2p) == (0, 898, 1463, 2048), "unexpected windows"
    rows = np.arange(N)
    src = np.empty((N, J), dtype=np.int32)
    for j in range(J):
        src[:, j] = rows * J + j
    for j in range(x2n // D):
        src[:, j] = perm_n * J + j
    for j in range(-(-x1p // D), J):
        src[:, j] = perm_p * J + j
    b7 = (perm_n * J + 7).astype(np.int32)
    b11 = (perm_p * J + 11).astype(np.int32)
    return (src.reshape(NW, ROWS_PER_W * J // D, D),
            b7.reshape(NW, NCH, CH), b11.reshape(NW, NCH, CH))


_SRC2, _B7, _B11 = _build_indices()


@functools.partial(
    pl.kernel,
    mesh=plsc.VectorSubcoreMesh(core_axis_name="c", subcore_axis_name="s"),
    out_type=jax.ShapeDtypeStruct((N * J, D), jnp.float32),
    scratch_types=[
        pltpu.VMEM((ROWS_PER_W * J // D, D), jnp.int32),
        pltpu.VMEM((NCH, CH), jnp.int32),
        pltpu.VMEM((NCH, CH), jnp.int32),
        pltpu.VMEM((CH * J, D), jnp.float32),
        pltpu.VMEM((CH, D), jnp.float32),
        pltpu.VMEM((CH, D), jnp.float32),
        pltpu.SemaphoreType.DMA,
    ],
)
def _gather_kernel(qf, src2, src7, src11, out, idx_v, idx7_v, idx11_v,
                   chunk_v, b7_v, b11_v, sem):
    wid = lax.axis_index("s") * NC + lax.axis_index("c")
    lane = lax.iota(jnp.int32, 16)
    m7 = lane < 2
    m11 = lane >= 7

    pltpu.sync_copy(src2.at[wid], idx_v)
    pltpu.sync_copy(src7.at[wid], idx7_v)
    pltpu.sync_copy(src11.at[wid], idx11_v)
    kpc = CH * J // D

    def chunk_body(c, carry):
        rbase = wid * ROWS_PER_W + c * CH
        fbase = rbase * J
        copies = [
            pltpu.async_copy(qf.at[idx_v.at[c * kpc + k]],
                             chunk_v.at[pl.ds(k * D, D)], sem)
            for k in range(kpc)
        ]
        copies.append(pltpu.async_copy(qf.at[idx7_v.at[c]], b7_v, sem))
        copies.append(pltpu.async_copy(qf.at[idx11_v.at[c]], b11_v, sem))
        for cp in copies:
            cp.wait()
        for r in range(CH):
            r7 = r * J + 7
            v = jnp.where(m7, b7_v[r, pl.ds(0, 16)], chunk_v[r7, pl.ds(0, 16)])
            chunk_v[r7, pl.ds(0, 16)] = v
            r11 = r * J + 11
            v = jnp.where(m11, b11_v[r, pl.ds(48, 16)],
                          chunk_v[r11, pl.ds(48, 16)])
            chunk_v[r11, pl.ds(48, 16)] = v
            for t in range(4, 8):
                chunk_v[r11, pl.ds(t * 16, 16)] = b11_v[r, pl.ds(t * 16, 16)]
        pltpu.sync_copy(chunk_v, out.at[pl.ds(fbase, CH * J)])
        return carry

    lax.fori_loop(0, NCH, chunk_body, 0)


def kernel(query):
    qf = query.reshape(N * J, D)
    out = _gather_kernel(qf, jnp.asarray(_SRC2), jnp.asarray(_B7),
                         jnp.asarray(_B11))
    q = out.reshape(N, W)
    return (q, q, q)


# speedup vs baseline: 179.2933x; 1.0254x over previous
import functools

import jax
import jax.numpy as jnp
import numpy as np
from jax import lax
from jax.experimental import pallas as pl
from jax.experimental.pallas import tpu as pltpu
from jax.experimental.pallas import tpu_sc as plsc

N = 16384
W = 2048
D = 128
J = W // D
NC, NS = 2, 16
NW = NC * NS
ROWS_PER_W = N // NW
CH = 16
NCH = ROWS_PER_W // CH
KPC = CH * J // D


def _build_indices():
    rng = np.random.RandomState(0)
    perm_p = rng.permutation(N)
    x_p = int(rng.randint(W))
    perm_n = rng.permutation(N)
    x_n = int(rng.randint(W))
    x1p = int(np.clip(x_p - 300, 0, W))
    x2p = int(np.clip(x_p + 300, 0, W))
    x1n = int(np.clip(x_n - 500, 0, W))
    x2n = int(np.clip(x_n + 500, 0, W))
    assert (x1n, x2n, x1p, x2p) == (0, 898, 1463, 2048), "unexpected windows"
    rows = np.arange(N)
    src = np.empty((N, J), dtype=np.int32)
    for j in range(J):
        src[:, j] = rows * J + j
    for j in range(x2n // D):
        src[:, j] = perm_n * J + j
    for j in range(-(-x1p // D), J):
        src[:, j] = perm_p * J + j
    b7 = (perm_n * J + 7).astype(np.int32)
    b11 = (perm_p * J + 11).astype(np.int32)
    return (src.reshape(NW, ROWS_PER_W * J // D, D),
            b7.reshape(NW, NCH, CH), b11.reshape(NW, NCH, CH))


_SRC2, _B7, _B11 = _build_indices()


@functools.partial(
    pl.kernel,
    mesh=plsc.VectorSubcoreMesh(core_axis_name="c", subcore_axis_name="s"),
    out_type=jax.ShapeDtypeStruct((N * J, D), jnp.float32),
    scratch_types=[
        pltpu.VMEM((ROWS_PER_W * J // D, D), jnp.int32),
        pltpu.VMEM((NCH, CH), jnp.int32),
        pltpu.VMEM((NCH, CH), jnp.int32),
        pltpu.VMEM((CH * J, D), jnp.float32),
        pltpu.VMEM((CH * J, D), jnp.float32),
        pltpu.VMEM((CH, D), jnp.float32),
        pltpu.VMEM((CH, D), jnp.float32),
        pltpu.VMEM((CH, D), jnp.float32),
        pltpu.VMEM((CH, D), jnp.float32),
        pltpu.SemaphoreType.DMA,
        pltpu.SemaphoreType.DMA,
    ],
)
def _gather_kernel(qf, src2, src7, src11, out, idx_v, idx7_v, idx11_v,
                   buf_a, buf_b, b7_a, b7_b, b11_a, b11_b, sem_a, sem_b):
    wid = lax.axis_index("s") * NC + lax.axis_index("c")
    lane = lax.iota(jnp.int32, 16)
    m7 = lane < 2
    m11 = lane >= 7

    pltpu.sync_copy(src2.at[wid], idx_v)
    pltpu.sync_copy(src7.at[wid], idx7_v)
    pltpu.sync_copy(src11.at[wid], idx11_v)

    def issue(c, buf, b7v, b11v, sem):
        for k in range(KPC):
            pltpu.async_copy(qf.at[idx_v.at[c * KPC + k]],
                             buf.at[pl.ds(k * D, D)], sem)
        pltpu.async_copy(qf.at[idx7_v.at[c]], b7v, sem)
        pltpu.async_copy(qf.at[idx11_v.at[c]], b11v, sem)

    def drain(buf, b7v, b11v, sem):
        pltpu.make_async_copy(qf.at[pl.ds(0, CH * J)], buf, sem).wait()
        pltpu.make_async_copy(qf.at[pl.ds(0, CH)], b7v, sem).wait()
        pltpu.make_async_copy(qf.at[pl.ds(0, CH)], b11v, sem).wait()

    def blend(buf, b7v, b11v):
        for r in range(CH):
            r7 = r * J + 7
            v = jnp.where(m7, b7v[r, pl.ds(0, 16)], buf[r7, pl.ds(0, 16)])
            buf[r7, pl.ds(0, 16)] = v
            r11 = r * J + 11
            v = jnp.where(m11, b11v[r, pl.ds(48, 16)], buf[r11, pl.ds(48, 16)])
            buf[r11, pl.ds(48, 16)] = v
            for t in range(4, 8):
                buf[r11, pl.ds(t * 16, 16)] = b11v[r, pl.ds(t * 16, 16)]

    def store(c, buf):
        fbase = (wid * NCH + c) * CH * J
        pltpu.sync_copy(buf, out.at[pl.ds(fbase, CH * J)])

    issue(0, buf_a, b7_a, b11_a, sem_a)

    def pair_body(p, carry):
        c = 2 * p
        issue(c + 1, buf_b, b7_b, b11_b, sem_b)
        drain(buf_a, b7_a, b11_a, sem_a)
        blend(buf_a, b7_a, b11_a)
        store(c, buf_a)

        @pl.when(p < NCH // 2 - 1)
        def _():
            issue(c + 2, buf_a, b7_a, b11_a, sem_a)

        drain(buf_b, b7_b, b11_b, sem_b)
        blend(buf_b, b7_b, b11_b)
        store(c + 1, buf_b)
        return carry

    lax.fori_loop(0, NCH // 2, pair_body, 0)


def kernel(query):
    qf = query.reshape(N * J, D)
    out = _gather_kernel(qf, jnp.asarray(_SRC2), jnp.asarray(_B7),
                         jnp.asarray(_B11))
    q = out.reshape(N, W)
    return (q, q, q)


# speedup vs baseline: 349.9691x; 1.9519x over previous
import functools

import jax
import jax.numpy as jnp
import numpy as np
from jax import lax
from jax.experimental import pallas as pl
from jax.experimental.pallas import tpu as pltpu
from jax.experimental.pallas import tpu_sc as plsc

N = 16384
W = 2048
NC, NS = 2, 16
NW = NC * NS
ROWS_PER_W = N // NW
CH = 16
NCH = ROWS_PER_W // CH

SPAN_N = 1024
ID_LO, ID_HI = 896, 1536
SPAN_P_LO = 1408
SPAN_P = W - SPAN_P_LO


def _build_indices():
    rng = np.random.RandomState(0)
    perm_p = rng.permutation(N)
    x_p = int(rng.randint(W))
    perm_n = rng.permutation(N)
    x_n = int(rng.randint(W))
    x1p = int(np.clip(x_p - 300, 0, W))
    x2p = int(np.clip(x_p + 300, 0, W))
    x1n = int(np.clip(x_n - 500, 0, W))
    x2n = int(np.clip(x_n + 500, 0, W))
    assert (x1n, x2n, x1p, x2p) == (0, 898, 1463, 2048), "unexpected windows"
    return (perm_n.astype(np.int32).reshape(NW, NCH, CH),
            perm_p.astype(np.int32).reshape(NW, NCH, CH))


_IDXN, _IDXP = _build_indices()


@functools.partial(
    pl.kernel,
    mesh=plsc.VectorSubcoreMesh(core_axis_name="c", subcore_axis_name="s"),
    out_type=jax.ShapeDtypeStruct((N, W), jnp.float32),
    scratch_types=[
        pltpu.VMEM((NCH, CH), jnp.int32),
        pltpu.VMEM((NCH, CH), jnp.int32),
        pltpu.VMEM((CH, SPAN_N), jnp.float32),
        pltpu.VMEM((CH, SPAN_N), jnp.float32),
        pltpu.VMEM((CH, SPAN_P), jnp.float32),
        pltpu.VMEM((CH, SPAN_P), jnp.float32),
        pltpu.VMEM((CH, ID_HI - ID_LO), jnp.float32),
        pltpu.VMEM((CH, ID_HI - ID_LO), jnp.float32),
        pltpu.SemaphoreType.DMA,
        pltpu.SemaphoreType.DMA,
    ],
)
def _gather_kernel(q, idxn, idxp, out, idxn_v, idxp_v, bn_a, bn_b,
                   bp_a, bp_b, bid_a, bid_b, sem_a, sem_b):
    wid = lax.axis_index("s") * NC + lax.axis_index("c")
    lane = lax.iota(jnp.int32, 16)
    m_n = lane >= 2
    m_p = lane < 7

    pltpu.sync_copy(idxn.at[wid], idxn_v)
    pltpu.sync_copy(idxp.at[wid], idxp_v)

    def issue(c, bn, bp, bid, sem):
        rbase = wid * ROWS_PER_W + c * CH
        pltpu.async_copy(q.at[idxn_v.at[c], pl.ds(0, SPAN_N)], bn, sem)
        pltpu.async_copy(q.at[idxp_v.at[c], pl.ds(SPAN_P_LO, SPAN_P)], bp, sem)
        pltpu.async_copy(q.at[pl.ds(rbase, CH), pl.ds(ID_LO, ID_HI - ID_LO)],
                         bid, sem)

    def drain(bn, bp, bid, sem):
        pltpu.make_async_copy(q.at[pl.ds(0, CH), pl.ds(0, SPAN_N)], bn, sem).wait()
        pltpu.make_async_copy(q.at[pl.ds(0, CH), pl.ds(0, SPAN_P)], bp, sem).wait()
        pltpu.make_async_copy(q.at[pl.ds(0, CH), pl.ds(0, ID_HI - ID_LO)],
                              bid, sem).wait()

    def blend(bn, bp, bid):
        for r in range(CH):
            v = jnp.where(m_n, bid[r, pl.ds(0, 16)], bn[r, pl.ds(896, 16)])
            bn[r, pl.ds(896, 16)] = v
            for t in range(1, 8):
                bn[r, pl.ds(896 + 16 * t, 16)] = bid[r, pl.ds(16 * t, 16)]
            for t in range(3):
                bp[r, pl.ds(16 * t, 16)] = bid[r, pl.ds(512 + 16 * t, 16)]
            v = jnp.where(m_p, bid[r, pl.ds(560, 16)], bp[r, pl.ds(48, 16)])
            bp[r, pl.ds(48, 16)] = v

    def store(c, bn, bp, bid):
        rbase = wid * ROWS_PER_W + c * CH
        rows = pl.ds(rbase, CH)
        pltpu.sync_copy(bn, out.at[rows, pl.ds(0, SPAN_N)])
        pltpu.sync_copy(bid.at[:, pl.ds(128, SPAN_P_LO - SPAN_N)],
                        out.at[rows, pl.ds(SPAN_N, SPAN_P_LO - SPAN_N)])
        pltpu.sync_copy(bp, out.at[rows, pl.ds(SPAN_P_LO, SPAN_P)])

    issue(0, bn_a, bp_a, bid_a, sem_a)

    def pair_body(p, carry):
        c = 2 * p
        issue(c + 1, bn_b, bp_b, bid_b, sem_b)
        drain(bn_a, bp_a, bid_a, sem_a)
        blend(bn_a, bp_a, bid_a)
        store(c, bn_a, bp_a, bid_a)

        @pl.when(p < NCH // 2 - 1)
        def _():
            issue(c + 2, bn_a, bp_a, bid_a, sem_a)

        drain(bn_b, bp_b, bid_b, sem_b)
        blend(bn_b, bp_b, bid_b)
        store(c + 1, bn_b, bp_b, bid_b)
        return carry

    lax.fori_loop(0, NCH // 2, pair_body, 0)


def kernel(query):
    out = _gather_kernel(query, jnp.asarray(_IDXN), jnp.asarray(_IDXP))
    return (out, out, out)
